# baseline (speedup 1.0000x reference)
import functools
import numpy as np
import jax
import jax.numpy as jnp
from jax import lax
from jax.experimental import pallas as pl
from jax.experimental.pallas import tpu as pltpu
from jax.experimental.pallas import tpu_sc as plsc

VOCAB = 100000
LATENT = 64
NUM_EV = 200
WORDS = 50

NC = 2
NS = 16
NW = NC * NS
J_PER = LATENT // NW
NSENT = 208
NGRP = NSENT // 16


def _sc_embed(tq, te, idxT, evc_out, evf_out, row_v, idx_v,
              ov0, ov1, ov2, ov3, sem, semo):
    wid = lax.axis_index("s") * NC + lax.axis_index("c")
    zero = jnp.zeros((16,), jnp.float32)
    inv_w = jnp.float32(1.0 / WORDS)
    inv_l = jnp.float32(1.0 / LATENT)

    pltpu.sync_copy(idxT, idx_v)

    for tbl, out, ova, ovb in ((tq, evc_out, ov0, ov1),
                               (te, evf_out, ov2, ov3)):
        def jo_body(jo, carry, tbl=tbl, out=out, ova=ova, ovb=ovb):
            j = wid * J_PER + jo
            pltpu.async_copy(tbl.at[j], row_v, sem).wait()
            jf = jnp.full((16,), j, jnp.float32) * inv_l

            def body(w, accs, jf=jf):
                wf = jnp.full((16,), w, jnp.float32) * inv_w
                pe = (1.0 - wf) + jf * (2.0 * wf - 1.0)
                new = []
                for g in range(NGRP):
                    iv = idx_v[w, pl.ds(g * 16, 16)]
                    vals = plsc.load_gather(row_v, [iv])
                    new.append(accs[g] + vals * pe)
                return tuple(new)

            accs = lax.fori_loop(0, WORDS, body, (zero,) * NGRP)
            for g in range(NGRP):
                ova[pl.ds(g * 16, 16)] = accs[g]
            pltpu.sync_copy(ova, out.at[j])
            return carry

        lax.fori_loop(0, J_PER, jo_body, 0)


@jax.jit
def _sc_call(tq, te, idxT):
    mesh = plsc.VectorSubcoreMesh(core_axis_name="c", subcore_axis_name="s",
                                  num_cores=NC, num_subcores=NS)
    f32 = jnp.float32
    return pl.kernel(
        _sc_embed,
        out_type=(
            jax.ShapeDtypeStruct((LATENT, NSENT), f32),
            jax.ShapeDtypeStruct((LATENT, NSENT), f32),
        ),
        mesh=mesh,
        scratch_types=(
            pltpu.VMEM((VOCAB,), f32),
            pltpu.VMEM((WORDS, NSENT), jnp.int32),
            pltpu.VMEM((NSENT,), f32),
            pltpu.VMEM((NSENT,), f32),
            pltpu.VMEM((NSENT,), f32),
            pltpu.VMEM((NSENT,), f32),
            pltpu.SemaphoreType.DMA,
            pltpu.SemaphoreType.DMA,
        ),
        compiler_params=pltpu.CompilerParams(use_tc_tiling_on_sc=True,
                                             needs_layout_passes=False),
    )(tq, te, idxT)


BVC = 8192
NBLK = (VOCAB + BVC - 1) // BVC


def _tc_body(evc_ref, evf_ref, t1_ref, t2_ref, fcw_ref, fcb_ref, out_ref,
             logit_s, feat_s, ms_s):
    j = pl.program_id(0)
    dnum_00 = (((0,), (0,)), ((), ()))
    dnum_11 = (((1,), (1,)), ((), ()))

    @pl.when(j == 0)
    def _():
        evc = evc_ref[...]
        evc_e = evc + t2_ref[...]
        evf_e = evf_ref[...] + t1_ref[...]
        lane = lax.broadcasted_iota(jnp.int32, (1, NSENT), 1)
        zf = lax.dot_general(evc, evc_e, dnum_00,
                             preferred_element_type=jnp.float32)
        z = zf[NUM_EV:NUM_EV + 1, :]
        z = jnp.where(lane < NUM_EV, z, -jnp.inf)
        z = z - jnp.max(z)
        e = jnp.exp(z)
        w = e / jnp.sum(e)
        onehot = (lane == NUM_EV).astype(jnp.float32)
        feat_s[...] = (
            lax.dot_general(w, evf_e, dnum_11,
                            preferred_element_type=jnp.float32)
            + lax.dot_general(onehot, evc, dnum_11,
                              preferred_element_type=jnp.float32))
        ms_s[0] = -jnp.inf

    @pl.when(j < NBLK)
    def _():
        f = feat_s[...]
        l = lax.dot_general(f, fcw_ref[...], (((1,), (0,)), ((), ())),
                            preferred_element_type=jnp.float32)
        l = l + fcb_ref[0]
        col = j * BVC + lax.broadcasted_iota(jnp.int32, (1, BVC), 1)
        l = jnp.where(col < VOCAB, l, -1e30)
        logit_s[pl.ds(j, 1), :] = l
        ms_s[0] = jnp.maximum(ms_s[0], jnp.max(l))

    @pl.when(j == NBLK)
    def _():
        e = jnp.exp(logit_s[...] - ms_s[0])
        out_ref[...] = e * (1.0 / jnp.sum(e))


@jax.jit
def _tc_call(evcT, evfT, t1T, t2T, fc_wT, fc_b):
    f32 = jnp.float32
    return pl.pallas_call(
        _tc_body,
        grid=(NBLK + 1,),
        in_specs=[
            pl.BlockSpec((LATENT, NSENT), lambda j: (0, 0)),
            pl.BlockSpec((LATENT, NSENT), lambda j: (0, 0)),
            pl.BlockSpec((LATENT, NSENT), lambda j: (0, 0)),
            pl.BlockSpec((LATENT, NSENT), lambda j: (0, 0)),
            pl.BlockSpec((LATENT, BVC),
                         lambda j: (0, jnp.minimum(j, NBLK - 1))),
            pl.BlockSpec((1, 1, BVC),
                         lambda j: (jnp.minimum(j, NBLK - 1), 0, 0)),
        ],
        out_specs=pl.BlockSpec((NBLK, BVC), lambda j: (0, 0)),
        out_shape=jax.ShapeDtypeStruct((NBLK, BVC), f32),
        scratch_shapes=[
            pltpu.VMEM((NBLK, BVC), f32),
            pltpu.VMEM((1, LATENT), f32),
            pltpu.SMEM((2,), f32),
        ],
        compiler_params=pltpu.CompilerParams(
            dimension_semantics=("arbitrary",),
        ),
    )(evcT, evfT, t1T, t2T, fc_wT, fc_b)


def kernel(evidence, question, question_table, evidence_table,
           temporal_enc1, temporal_enc2, fc_w, fc_b):
    ev_T = evidence.astype(jnp.int32).T
    q_T = question.astype(jnp.int32).T
    pad = jnp.zeros((WORDS, NSENT - NUM_EV - 1), jnp.int32)
    idxT = jnp.concatenate([ev_T, q_T, pad], axis=1)

    t1p = jnp.pad(temporal_enc1.T, ((0, 0), (0, NSENT - NUM_EV)))
    t2p = jnp.pad(temporal_enc2.T, ((0, 0), (0, NSENT - NUM_EV)))
    fcb2 = jnp.pad(fc_b, (0, NBLK * BVC - VOCAB),
                   constant_values=-1e30).reshape(NBLK, 1, BVC)

    evcT, evfT = _sc_call(question_table.T, evidence_table.T, idxT)
    probs2 = _tc_call(evcT, evfT, t1p, t2p, fc_w.T, fcb2)
    return probs2.reshape(-1)[:VOCAB]


# speedup vs baseline: 1.0126x; 1.0126x over previous
import functools
import numpy as np
import jax
import jax.numpy as jnp
from jax import lax
from jax.experimental import pallas as pl
from jax.experimental.pallas import tpu as pltpu
from jax.experimental.pallas import tpu_sc as plsc

VOCAB = 100000
LATENT = 64
NUM_EV = 200
WORDS = 50

NC = 2
NS = 16
NW = NC * NS
J_PER = LATENT // NW
NSENT = 216
WINDOWS = tuple(g * 16 for g in range(11)) + (176, 184)


def _sc_embed(tq, te, evT, q, evc_out, evf_out, row_v, ev_v, qv,
              ov0, ov1, ov2, ov3, sem, semo):
    wid = lax.axis_index("s") * NC + lax.axis_index("c")
    zero = jnp.zeros((16,), jnp.float32)
    zero_i = jnp.zeros((16,), jnp.int32)
    inv_w = jnp.float32(1.0 / WORDS)
    inv_l = jnp.float32(1.0 / LATENT)

    passes = ((tq, evc_out, 0, ov0, True), (tq, evc_out, 1, ov1, True),
              (te, evf_out, 0, ov2, False), (te, evf_out, 1, ov3, False))
    cp = pltpu.async_copy(tq.at[wid * J_PER], row_v, sem)
    pltpu.sync_copy(evT, ev_v)
    pltpu.sync_copy(q, qv)

    out_cps = []
    for p, (tbl, out, jo, ov, with_q) in enumerate(passes):
        j = wid * J_PER + jo
        cp.wait()
        jf = jnp.full((16,), j, jnp.float32) * inv_l

        def body(w, accs, jf=jf, with_q=with_q):
            wf = jnp.full((16,), w, jnp.float32) * inv_w
            pe = (1.0 - wf) + jf * (2.0 * wf - 1.0)
            new = []
            for g, base in enumerate(WINDOWS):
                iv = ev_v[w, pl.ds(base, 16)]
                vals = plsc.load_gather(row_v, [iv])
                new.append(accs[g] + vals * pe)
            if with_q:
                qi = plsc.load_gather(qv, [zero_i, jnp.full((16,), w, jnp.int32)])
                qvals = plsc.load_gather(row_v, [qi])
                new.append(accs[-1] + qvals * pe)
            return tuple(new)

        n_acc = len(WINDOWS) + (1 if with_q else 0)
        accs = lax.fori_loop(0, WORDS, body, (zero,) * n_acc)
        if p < 3:
            ntbl, _, njo, _, _ = passes[p + 1]
            cp = pltpu.async_copy(ntbl.at[wid * J_PER + njo], row_v, sem)
        for g, base in enumerate(WINDOWS):
            ov[pl.ds(base, 16)] = accs[g]
        lane = lax.broadcasted_iota(jnp.int32, (16,), 0)
        if with_q:
            ov[pl.ds(200, 16)] = jnp.where(lane == 0, accs[-1], 0.0)
        else:
            ov[pl.ds(200, 16)] = zero
        out_cps.append(pltpu.async_copy(ov, out.at[j], semo))
    for c in out_cps:
        c.wait()


@jax.jit
def _sc_call(tq, te, evT, q):
    mesh = plsc.VectorSubcoreMesh(core_axis_name="c", subcore_axis_name="s",
                                  num_cores=NC, num_subcores=NS)
    f32 = jnp.float32
    return pl.kernel(
        _sc_embed,
        out_type=(
            jax.ShapeDtypeStruct((LATENT, NSENT), f32),
            jax.ShapeDtypeStruct((LATENT, NSENT), f32),
        ),
        mesh=mesh,
        scratch_types=(
            pltpu.VMEM((VOCAB,), f32),
            pltpu.VMEM((WORDS, NUM_EV), jnp.int32),
            pltpu.VMEM((1, WORDS), jnp.int32),
            pltpu.VMEM((NSENT,), f32),
            pltpu.VMEM((NSENT,), f32),
            pltpu.VMEM((NSENT,), f32),
            pltpu.VMEM((NSENT,), f32),
            pltpu.SemaphoreType.DMA,
            pltpu.SemaphoreType.DMA,
        ),
        compiler_params=pltpu.CompilerParams(use_tc_tiling_on_sc=True,
                                             needs_layout_passes=False),
    )(tq, te, evT, q)


BVC = 8192
NBLK = (VOCAB + BVC - 1) // BVC


def _tc_body(evc_ref, evf_ref, t1_ref, t2_ref, fcw_ref, fcb_ref, out_ref,
             logit_s, feat_s, ms_s):
    j = pl.program_id(0)
    dnum_00 = (((0,), (0,)), ((), ()))
    dnum_11 = (((1,), (1,)), ((), ()))

    @pl.when(j == 0)
    def _():
        evc = evc_ref[...]
        evc_e = evc + t2_ref[...]
        evf_e = evf_ref[...] + t1_ref[...]
        lane = lax.broadcasted_iota(jnp.int32, (1, NSENT), 1)
        zf = lax.dot_general(evc, evc_e, dnum_00,
                             preferred_element_type=jnp.float32)
        z = zf[NUM_EV:NUM_EV + 1, :]
        z = jnp.where(lane < NUM_EV, z, -jnp.inf)
        z = z - jnp.max(z)
        e = jnp.exp(z)
        w = e / jnp.sum(e)
        onehot = (lane == NUM_EV).astype(jnp.float32)
        feat_s[...] = (
            lax.dot_general(w, evf_e, dnum_11,
                            preferred_element_type=jnp.float32)
            + lax.dot_general(onehot, evc, dnum_11,
                              preferred_element_type=jnp.float32))
        ms_s[0] = -jnp.inf

    @pl.when(j < NBLK)
    def _():
        f = feat_s[...]
        l = lax.dot_general(f, fcw_ref[...], (((1,), (0,)), ((), ())),
                            preferred_element_type=jnp.float32)
        l = l + fcb_ref[0]
        col = j * BVC + lax.broadcasted_iota(jnp.int32, (1, BVC), 1)
        l = jnp.where(col < VOCAB, l, -1e30)
        logit_s[pl.ds(j, 1), :] = l
        ms_s[0] = jnp.maximum(ms_s[0], jnp.max(l))

    @pl.when(j == NBLK)
    def _():
        e = jnp.exp(logit_s[...] - ms_s[0])
        out_ref[...] = e * (1.0 / jnp.sum(e))


@jax.jit
def _tc_call(evcT, evfT, t1T, t2T, fc_wT, fc_b):
    f32 = jnp.float32
    return pl.pallas_call(
        _tc_body,
        grid=(NBLK + 1,),
        in_specs=[
            pl.BlockSpec((LATENT, NSENT), lambda j: (0, 0)),
            pl.BlockSpec((LATENT, NSENT), lambda j: (0, 0)),
            pl.BlockSpec((LATENT, NSENT), lambda j: (0, 0)),
            pl.BlockSpec((LATENT, NSENT), lambda j: (0, 0)),
            pl.BlockSpec((LATENT, BVC),
                         lambda j: (0, jnp.minimum(j, NBLK - 1))),
            pl.BlockSpec((1, 1, BVC),
                         lambda j: (jnp.minimum(j, NBLK - 1), 0, 0)),
        ],
        out_specs=pl.BlockSpec((NBLK, BVC), lambda j: (0, 0)),
        out_shape=jax.ShapeDtypeStruct((NBLK, BVC), f32),
        scratch_shapes=[
            pltpu.VMEM((NBLK, BVC), f32),
            pltpu.VMEM((1, LATENT), f32),
            pltpu.SMEM((2,), f32),
        ],
        compiler_params=pltpu.CompilerParams(
            dimension_semantics=("arbitrary",),
        ),
    )(evcT, evfT, t1T, t2T, fc_wT, fc_b)


def kernel(evidence, question, question_table, evidence_table,
           temporal_enc1, temporal_enc2, fc_w, fc_b):
    ev_T = evidence.astype(jnp.int32).T
    q_i = question.astype(jnp.int32)

    t1p = jnp.pad(temporal_enc1.T, ((0, 0), (0, NSENT - NUM_EV)))
    t2p = jnp.pad(temporal_enc2.T, ((0, 0), (0, NSENT - NUM_EV)))
    fcb2 = jnp.pad(fc_b, (0, NBLK * BVC - VOCAB),
                   constant_values=-1e30).reshape(NBLK, 1, BVC)

    evcT, evfT = _sc_call(question_table.T, evidence_table.T, ev_T, q_i)
    probs2 = _tc_call(evcT, evfT, t1p, t2p, fc_w.T, fcb2)
    return probs2.reshape(-1)[:VOCAB]


# speedup vs baseline: 1.0696x; 1.0562x over previous
import functools
import numpy as np
import jax
import jax.numpy as jnp
from jax import lax
from jax.experimental import pallas as pl
from jax.experimental.pallas import tpu as pltpu
from jax.experimental.pallas import tpu_sc as plsc

VOCAB = 100000
LATENT = 64
NUM_EV = 200
WORDS = 50

NC = 2
NS = 16
NW = NC * NS
J_PER = LATENT // NW
NSENT = 216
WINDOWS = tuple(g * 16 for g in range(11)) + (176, 184)


def _sc_embed(tq, te, evT, q, evc_out, evf_out, row_v, ev_v, qv,
              ov0, ov1, ov2, ov3, sem, semo):
    wid = lax.axis_index("s") * NC + lax.axis_index("c")
    zero = jnp.zeros((16,), jnp.float32)
    zero_i = jnp.zeros((16,), jnp.int32)
    inv_w = jnp.float32(1.0 / WORDS)
    inv_l = jnp.float32(1.0 / LATENT)

    passes = ((tq, evc_out, 0, ov0, True), (tq, evc_out, 1, ov1, True),
              (te, evf_out, 0, ov2, False), (te, evf_out, 1, ov3, False))
    cp = pltpu.async_copy(tq.at[wid * J_PER], row_v, sem)
    pltpu.sync_copy(evT, ev_v)
    pltpu.sync_copy(q, qv)

    out_cps = []
    for p, (tbl, out, jo, ov, with_q) in enumerate(passes):
        j = wid * J_PER + jo
        cp.wait()
        jf = jnp.full((16,), j, jnp.float32) * inv_l

        def body(w, accs, jf=jf, with_q=with_q):
            wf = jnp.full((16,), w, jnp.float32) * inv_w
            pe = (1.0 - wf) + jf * (2.0 * wf - 1.0)
            new = []
            for g, base in enumerate(WINDOWS):
                iv = ev_v[w, pl.ds(base, 16)]
                vals = plsc.load_gather(row_v, [iv])
                new.append(accs[g] + vals * pe)
            if with_q:
                qi = plsc.load_gather(qv, [zero_i, jnp.full((16,), w, jnp.int32)])
                qvals = plsc.load_gather(row_v, [qi])
                new.append(accs[-1] + qvals * pe)
            return tuple(new)

        n_acc = len(WINDOWS) + (1 if with_q else 0)
        accs = lax.fori_loop(0, WORDS, body, (zero,) * n_acc)
        if p < 3:
            ntbl, _, njo, _, _ = passes[p + 1]
            cp = pltpu.async_copy(ntbl.at[wid * J_PER + njo], row_v, sem)
        for g, base in enumerate(WINDOWS):
            ov[pl.ds(base, 16)] = accs[g]
        lane = lax.broadcasted_iota(jnp.int32, (16,), 0)
        if with_q:
            ov[pl.ds(200, 16)] = jnp.where(lane == 0, accs[-1], 0.0)
        else:
            ov[pl.ds(200, 16)] = zero
        out_cps.append(pltpu.async_copy(ov, out.at[j], semo))
    for c in out_cps:
        c.wait()


@jax.jit
def _sc_call(tq, te, evT, q):
    mesh = plsc.VectorSubcoreMesh(core_axis_name="c", subcore_axis_name="s",
                                  num_cores=NC, num_subcores=NS)
    f32 = jnp.float32
    return pl.kernel(
        _sc_embed,
        out_type=(
            jax.ShapeDtypeStruct((LATENT, NSENT), f32),
            jax.ShapeDtypeStruct((LATENT, NSENT), f32),
        ),
        mesh=mesh,
        scratch_types=(
            pltpu.VMEM((VOCAB,), f32),
            pltpu.VMEM((WORDS, NUM_EV), jnp.int32),
            pltpu.VMEM((1, WORDS), jnp.int32),
            pltpu.VMEM((NSENT,), f32),
            pltpu.VMEM((NSENT,), f32),
            pltpu.VMEM((NSENT,), f32),
            pltpu.VMEM((NSENT,), f32),
            pltpu.SemaphoreType.DMA,
            pltpu.SemaphoreType.DMA,
        ),
        compiler_params=pltpu.CompilerParams(use_tc_tiling_on_sc=True,
                                             needs_layout_passes=False),
    )(tq, te, evT, q)


BVC = 16384
NBLK = (VOCAB + BVC - 1) // BVC


def _tc_body(evc_ref, evf_ref, t1_ref, t2_ref, fcw_ref, fcb_ref, out_ref,
             logit_s, feat_s, ms_s):
    j = pl.program_id(0)
    dnum_00 = (((0,), (0,)), ((), ()))
    dnum_11 = (((1,), (1,)), ((), ()))

    @pl.when(j == 0)
    def _():
        evc = evc_ref[...]
        evc_e = evc + t2_ref[...]
        evf_e = evf_ref[...] + t1_ref[...]
        lane = lax.broadcasted_iota(jnp.int32, (1, NSENT), 1)
        zf = lax.dot_general(evc, evc_e, dnum_00,
                             preferred_element_type=jnp.float32)
        z = zf[NUM_EV:NUM_EV + 1, :]
        z = jnp.where(lane < NUM_EV, z, -jnp.inf)
        z = z - jnp.max(z)
        e = jnp.exp(z)
        w = e / jnp.sum(e)
        onehot = (lane == NUM_EV).astype(jnp.float32)
        feat_s[...] = (
            lax.dot_general(w, evf_e, dnum_11,
                            preferred_element_type=jnp.float32)
            + lax.dot_general(onehot, evc, dnum_11,
                              preferred_element_type=jnp.float32))
        ms_s[0] = -jnp.inf

    @pl.when(j < NBLK)
    def _():
        f = feat_s[...]
        l = lax.dot_general(f, fcw_ref[...], (((1,), (0,)), ((), ())),
                            preferred_element_type=jnp.float32)
        l = l + fcb_ref[0]
        col = j * BVC + lax.broadcasted_iota(jnp.int32, (1, BVC), 1)
        l = jnp.where(col < VOCAB, l, -1e30)
        logit_s[pl.ds(j, 1), :] = l
        ms_s[0] = jnp.maximum(ms_s[0], jnp.max(l))

    @pl.when(j == NBLK)
    def _():
        e = jnp.exp(logit_s[...] - ms_s[0])
        out_ref[...] = e * (1.0 / jnp.sum(e))


@jax.jit
def _tc_call(evcT, evfT, t1T, t2T, fc_wT, fc_b):
    f32 = jnp.float32
    return pl.pallas_call(
        _tc_body,
        grid=(NBLK + 1,),
        in_specs=[
            pl.BlockSpec((LATENT, NSENT), lambda j: (0, 0)),
            pl.BlockSpec((LATENT, NSENT), lambda j: (0, 0)),
            pl.BlockSpec((LATENT, NSENT), lambda j: (0, 0)),
            pl.BlockSpec((LATENT, NSENT), lambda j: (0, 0)),
            pl.BlockSpec((LATENT, BVC),
                         lambda j: (0, jnp.minimum(j, NBLK - 1))),
            pl.BlockSpec((1, 1, BVC),
                         lambda j: (jnp.minimum(j, NBLK - 1), 0, 0)),
        ],
        out_specs=pl.BlockSpec((NBLK, BVC), lambda j: (0, 0)),
        out_shape=jax.ShapeDtypeStruct((NBLK, BVC), f32),
        scratch_shapes=[
            pltpu.VMEM((NBLK, BVC), f32),
            pltpu.VMEM((1, LATENT), f32),
            pltpu.SMEM((2,), f32),
        ],
        compiler_params=pltpu.CompilerParams(
            dimension_semantics=("arbitrary",),
        ),
    )(evcT, evfT, t1T, t2T, fc_wT, fc_b)


def kernel(evidence, question, question_table, evidence_table,
           temporal_enc1, temporal_enc2, fc_w, fc_b):
    ev_T = evidence.astype(jnp.int32).T
    q_i = question.astype(jnp.int32)

    t1p = jnp.pad(temporal_enc1.T, ((0, 0), (0, NSENT - NUM_EV)))
    t2p = jnp.pad(temporal_enc2.T, ((0, 0), (0, NSENT - NUM_EV)))
    fcb2 = jnp.pad(fc_b, (0, NBLK * BVC - VOCAB),
                   constant_values=-1e30).reshape(NBLK, 1, BVC)

    evcT, evfT = _sc_call(question_table.T, evidence_table.T, ev_T, q_i)
    probs2 = _tc_call(evcT, evfT, t1p, t2p, fc_w.T, fcb2)
    return probs2.reshape(-1)[:VOCAB]


# speedup vs baseline: 1.0845x; 1.0140x over previous
import functools
import numpy as np
import jax
import jax.numpy as jnp
from jax import lax
from jax.experimental import pallas as pl
from jax.experimental.pallas import tpu as pltpu
from jax.experimental.pallas import tpu_sc as plsc

VOCAB = 100000
LATENT = 64
NUM_EV = 200
WORDS = 50

NC = 2
NS = 16
NW = NC * NS
J_PER = LATENT // NW
NSENT = 216
WINDOWS = tuple(g * 16 for g in range(11)) + (176, 184)


def _sc_embed(tq, te, evT, q, evc_out, evf_out, row_v, ev_v, qv,
              ov0, ov1, ov2, ov3, sem, semo):
    wid = lax.axis_index("s") * NC + lax.axis_index("c")
    zero = jnp.zeros((16,), jnp.float32)
    zero_i = jnp.zeros((16,), jnp.int32)
    inv_w = jnp.float32(1.0 / WORDS)
    inv_l = jnp.float32(1.0 / LATENT)

    passes = ((tq, evc_out, 0, ov0, True), (tq, evc_out, 1, ov1, True),
              (te, evf_out, 0, ov2, False), (te, evf_out, 1, ov3, False))
    cp = pltpu.async_copy(tq.at[wid * J_PER], row_v, sem)
    pltpu.sync_copy(evT, ev_v)
    pltpu.sync_copy(q, qv)

    out_cps = []
    for p, (tbl, out, jo, ov, with_q) in enumerate(passes):
        j = wid * J_PER + jo
        cp.wait()
        jf = jnp.full((16,), j, jnp.float32) * inv_l

        def body(w, accs, jf=jf, with_q=with_q):
            wf = jnp.full((16,), w, jnp.float32) * inv_w
            pe = (1.0 - wf) + jf * (2.0 * wf - 1.0)
            new = []
            for g, base in enumerate(WINDOWS):
                iv = ev_v[w, pl.ds(base, 16)]
                vals = plsc.load_gather(row_v, [iv])
                new.append(accs[g] + vals * pe)
            if with_q:
                qi = plsc.load_gather(qv, [zero_i, jnp.full((16,), w, jnp.int32)])
                qvals = plsc.load_gather(row_v, [qi])
                new.append(accs[-1] + qvals * pe)
            return tuple(new)

        n_acc = len(WINDOWS) + (1 if with_q else 0)
        accs = lax.fori_loop(0, WORDS, body, (zero,) * n_acc)
        if p < 3:
            ntbl, _, njo, _, _ = passes[p + 1]
            cp = pltpu.async_copy(ntbl.at[wid * J_PER + njo], row_v, sem)
        for g, base in enumerate(WINDOWS):
            ov[pl.ds(base, 16)] = accs[g]
        lane = lax.broadcasted_iota(jnp.int32, (16,), 0)
        if with_q:
            ov[pl.ds(200, 16)] = jnp.where(lane == 0, accs[-1], 0.0)
        else:
            ov[pl.ds(200, 16)] = zero
        out_cps.append(pltpu.async_copy(ov, out.at[j], semo))
    for c in out_cps:
        c.wait()


@jax.jit
def _sc_call(tq, te, evT, q):
    mesh = plsc.VectorSubcoreMesh(core_axis_name="c", subcore_axis_name="s",
                                  num_cores=NC, num_subcores=NS)
    f32 = jnp.float32
    return pl.kernel(
        _sc_embed,
        out_type=(
            jax.ShapeDtypeStruct((LATENT, NSENT), f32),
            jax.ShapeDtypeStruct((LATENT, NSENT), f32),
        ),
        mesh=mesh,
        scratch_types=(
            pltpu.VMEM((VOCAB,), f32),
            pltpu.VMEM((WORDS, NUM_EV), jnp.int32),
            pltpu.VMEM((1, WORDS), jnp.int32),
            pltpu.VMEM((NSENT,), f32),
            pltpu.VMEM((NSENT,), f32),
            pltpu.VMEM((NSENT,), f32),
            pltpu.VMEM((NSENT,), f32),
            pltpu.SemaphoreType.DMA,
            pltpu.SemaphoreType.DMA,
        ),
        compiler_params=pltpu.CompilerParams(use_tc_tiling_on_sc=True,
                                             needs_layout_passes=False),
    )(tq, te, evT, q)


BVC = 32768
NBLK = (VOCAB + BVC - 1) // BVC


def _tc_body(evc_ref, evf_ref, t1_ref, t2_ref, fcw_ref, fcb_ref, out_ref,
             logit_s, feat_s, ms_s):
    j = pl.program_id(0)
    dnum_00 = (((0,), (0,)), ((), ()))
    dnum_11 = (((1,), (1,)), ((), ()))

    @pl.when(j == 0)
    def _():
        evc = evc_ref[...]
        evc_e = evc + t2_ref[...]
        evf_e = evf_ref[...] + t1_ref[...]
        lane = lax.broadcasted_iota(jnp.int32, (1, NSENT), 1)
        zf = lax.dot_general(evc, evc_e, dnum_00,
                             preferred_element_type=jnp.float32)
        z = zf[NUM_EV:NUM_EV + 1, :]
        z = jnp.where(lane < NUM_EV, z, -jnp.inf)
        z = z - jnp.max(z)
        e = jnp.exp(z)
        w = e / jnp.sum(e)
        onehot = (lane == NUM_EV).astype(jnp.float32)
        feat_s[...] = (
            lax.dot_general(w, evf_e, dnum_11,
                            preferred_element_type=jnp.float32)
            + lax.dot_general(onehot, evc, dnum_11,
                              preferred_element_type=jnp.float32))
        ms_s[0] = -jnp.inf

    @pl.when(j < NBLK)
    def _():
        f = feat_s[...]
        l = lax.dot_general(f, fcw_ref[...], (((1,), (0,)), ((), ())),
                            preferred_element_type=jnp.float32)
        l = l + fcb_ref[0]
        col = j * BVC + lax.broadcasted_iota(jnp.int32, (1, BVC), 1)
        l = jnp.where(col < VOCAB, l, -1e30)
        logit_s[pl.ds(j, 1), :] = l
        ms_s[0] = jnp.maximum(ms_s[0], jnp.max(l))

    @pl.when(j == NBLK)
    def _():
        e = jnp.exp(logit_s[...] - ms_s[0])
        out_ref[...] = e * (1.0 / jnp.sum(e))


@jax.jit
def _tc_call(evcT, evfT, t1T, t2T, fc_wT, fc_b):
    f32 = jnp.float32
    return pl.pallas_call(
        _tc_body,
        grid=(NBLK + 1,),
        in_specs=[
            pl.BlockSpec((LATENT, NSENT), lambda j: (0, 0)),
            pl.BlockSpec((LATENT, NSENT), lambda j: (0, 0)),
            pl.BlockSpec((LATENT, NSENT), lambda j: (0, 0)),
            pl.BlockSpec((LATENT, NSENT), lambda j: (0, 0)),
            pl.BlockSpec((LATENT, BVC),
                         lambda j: (0, jnp.minimum(j, NBLK - 1))),
            pl.BlockSpec((1, 1, BVC),
                         lambda j: (jnp.minimum(j, NBLK - 1), 0, 0)),
        ],
        out_specs=pl.BlockSpec((NBLK, BVC), lambda j: (0, 0)),
        out_shape=jax.ShapeDtypeStruct((NBLK, BVC), f32),
        scratch_shapes=[
            pltpu.VMEM((NBLK, BVC), f32),
            pltpu.VMEM((1, LATENT), f32),
            pltpu.SMEM((2,), f32),
        ],
        compiler_params=pltpu.CompilerParams(
            dimension_semantics=("arbitrary",),
        ),
    )(evcT, evfT, t1T, t2T, fc_wT, fc_b)


def kernel(evidence, question, question_table, evidence_table,
           temporal_enc1, temporal_enc2, fc_w, fc_b):
    ev_T = evidence.astype(jnp.int32).T
    q_i = question.astype(jnp.int32)

    t1p = jnp.pad(temporal_enc1.T, ((0, 0), (0, NSENT - NUM_EV)))
    t2p = jnp.pad(temporal_enc2.T, ((0, 0), (0, NSENT - NUM_EV)))
    fcb2 = jnp.pad(fc_b, (0, NBLK * BVC - VOCAB),
                   constant_values=-1e30).reshape(NBLK, 1, BVC)

    evcT, evfT = _sc_call(question_table.T, evidence_table.T, ev_T, q_i)
    probs2 = _tc_call(evcT, evfT, t1p, t2p, fc_w.T, fcb2)
    return probs2.reshape(-1)[:VOCAB]
